# baseline (speedup 1.0000x reference)
import jax
import jax.numpy as jnp
from jax.experimental import pallas as pl
from jax.experimental.pallas import tpu as pltpu

ROW_BLK = 128
N_ROW = 4096
N_COL = 2048
N_BATCH = 4
N_K = N_ROW // ROW_BLK


def _argmin_body(x_ref, o_ref, mval, midx):
    k = pl.program_id(1)
    xb = x_ref[0]
    bm = jnp.min(xb, axis=0, keepdims=True)
    bi = jnp.argmin(xb, axis=0).astype(jnp.int32).reshape(1, N_COL) + k * ROW_BLK

    @pl.when(k == 0)
    def _init():
        mval[...] = bm
        midx[...] = bi

    @pl.when(k > 0)
    def _merge():
        better = bm < mval[...]
        mval[...] = jnp.where(better, bm, mval[...])
        midx[...] = jnp.where(better, bi, midx[...])

    @pl.when(k == N_K - 1)
    def _emit():
        o_ref[0] = midx[...]


def kernel(x):
    out = pl.pallas_call(
        _argmin_body,
        grid=(N_BATCH, N_K),
        in_specs=[pl.BlockSpec((1, ROW_BLK, N_COL), lambda b, k: (b, k, 0))],
        out_specs=pl.BlockSpec((1, 1, N_COL), lambda b, k: (b, 0, 0)),
        out_shape=jax.ShapeDtypeStruct((N_BATCH, 1, N_COL), jnp.int32),
        scratch_shapes=[
            pltpu.VMEM((1, N_COL), jnp.float32),
            pltpu.VMEM((1, N_COL), jnp.int32),
        ],
    )(x)
    return out.reshape(N_BATCH, N_COL).astype(jnp.int64)
